# baseline (speedup 1.0000x reference)
import functools
import math

import jax
import jax.numpy as jnp
from jax.experimental import pallas as pl
from jax.experimental.pallas import tpu as pltpu

_F32 = jnp.float32


def _ceil_to(v, m):
    return (v + m - 1) // m * m


def _mamba_body(hid_ref, w_in_ref, w_conv_ref, b_conv_ref, w_x_ref,
                w_dt_ref, b_dt_ref, neg_a_ref, d_ref, w_out_ref,
                out_ref,
                state_ref, ctail_ref, xs_ref, dl8_ref, dx8_ref,
                bb_ref, cc_ref, xa_ref, z_ref, y_ref, ya_ref,
                *, dip, d_state, d_conv, dtp, seg, tl, cpad):
    t_idx = pl.program_id(1)

    @pl.when(t_idx == 0)
    def _zero_carry():
        state_ref[...] = jnp.zeros_like(state_ref)
        ctail_ref[...] = jnp.zeros_like(ctail_ref)

    h = hid_ref[0]
    xz = jnp.dot(h, w_in_ref[...], preferred_element_type=_F32)
    x = xz[:, :dip]
    z_ref[...] = xz[:, dip:]

    lo = cpad - (d_conv - 1)
    xs_ref[lo:cpad, :] = ctail_ref[...]
    xs_ref[cpad:cpad + tl, :] = x
    ctail_ref[...] = xs_ref[cpad + tl - (d_conv - 1):cpad + tl, :]
    acc = b_conv_ref[...]
    for k in range(d_conv):
        acc = acc + xs_ref[lo + k:lo + k + tl, :] * w_conv_ref[k:k + 1, :]
    x_act = acc * jax.nn.sigmoid(acc)
    xa_ref[...] = x_act

    xd = jnp.dot(x_act.astype(w_x_ref.dtype), w_x_ref[...],
                 preferred_element_type=_F32)
    bb_ref[...] = xd[:, dtp:dtp + d_state]
    cc_ref[...] = xd[:, dtp + seg:dtp + seg + d_state]
    dt = jnp.dot(xd[:, :dtp].astype(w_dt_ref.dtype), w_dt_ref[...],
                 preferred_element_type=_F32)
    delta = jax.nn.softplus(dt + b_dt_ref[...])
    dl8_ref[...] = jnp.repeat(delta, 8, axis=0)
    dx8_ref[...] = jnp.repeat(delta * x_act, 8, axis=0)

    r_i = jax.lax.broadcasted_iota(jnp.int32, (d_state, d_state), 0)
    c_i = jax.lax.broadcasted_iota(jnp.int32, (d_state, d_state), 1)
    diag = (r_i == c_i).astype(_F32)

    hw = dip // 2

    def half_step(lo, na):
        def stepf(l, st):
            dlb = dl8_ref[pl.ds(l * 8, 8), lo:lo + hw]
            dxb = dx8_ref[pl.ds(l * 8, 8), lo:lo + hw]
            dl16 = jnp.concatenate([dlb, dlb], axis=0)
            dx16 = jnp.concatenate([dxb, dxb], axis=0)
            b_col = jnp.sum(diag * bb_ref[pl.ds(l, 1), :], axis=1,
                            keepdims=True)
            c_col = jnp.sum(diag * cc_ref[pl.ds(l, 1), :], axis=1,
                            keepdims=True)
            dA = jnp.exp2(dl16 * na)
            st = st * dA + b_col * dx16
            yp = st * c_col
            y_ref[pl.ds(l * 8, 8), lo:lo + hw] = (
                yp[0:8, :] + yp[8:16, :]).astype(y_ref.dtype)
            return st
        return stepf

    for lo in (0, hw):
        na = neg_a_ref[:, lo:lo + hw]
        st = jax.lax.fori_loop(0, tl, half_step(lo, na),
                               state_ref[:, lo:lo + hw], unroll=16)
        state_ref[:, lo:lo + hw] = st

    s_i = jax.lax.broadcasted_iota(jnp.int32, (8, 64), 0)
    l_i = jax.lax.broadcasted_iota(jnp.int32, (8, 64), 1)
    o8 = (l_i // 8 == s_i).astype(y_ref.dtype)
    xa = xa_ref[...]
    z = z_ref[...]
    gate = z * jax.nn.sigmoid(z)
    for g in range(tl // 8):
        yg = jnp.dot(o8, y_ref[g * 64:(g + 1) * 64, :],
                     preferred_element_type=_F32)
        r = slice(g * 8, (g + 1) * 8)
        ya_ref[r, :] = (yg + xa[r, :] * d_ref[...]) * gate[r, :]
    out = jnp.dot(ya_ref[...].astype(w_out_ref.dtype), w_out_ref[...],
                  preferred_element_type=_F32)
    out_ref[0] = out.astype(out_ref.dtype)


def _pack(a, rows, cols):
    a = a.astype(_F32)
    return jnp.pad(a, ((0, rows - a.shape[0]), (0, cols - a.shape[1])))


def kernel(hidden_states, w_in, w_conv, b_conv, w_x, w_dt, b_dt, A_log, D,
           w_out):
    d_state, d_conv, dt_rank = 16, 4, 48
    tl = 256
    cdt = jnp.bfloat16

    B, L, Dm = hidden_states.shape
    d_inner = w_in.shape[0] // 2

    Dp = _ceil_to(Dm, 128)
    dip = _ceil_to(d_inner, 128)
    tl = _ceil_to(max(16, min(tl, L)), 16)
    Lp = _ceil_to(L, tl)
    n_lt = Lp // tl
    dtp = _ceil_to(dt_rank, 128)
    seg = _ceil_to(d_state, 128)
    cpad = max(8, _ceil_to(d_conv - 1, 8))

    w_in_t = jnp.concatenate(
        [_pack(w_in[:d_inner].T, Dp, dip), _pack(w_in[d_inner:].T, Dp, dip)],
        axis=1).astype(cdt)
    w_conv_t = _pack(w_conv[:, 0, :].T, d_conv, dip)
    b_conv_p = _pack(b_conv[None, :], 1, dip)
    w_x_f = w_x.astype(_F32)
    w_x_t = jnp.concatenate(
        [_pack(w_x_f[:dt_rank].T, dip, dtp),
         _pack(w_x_f[dt_rank:dt_rank + d_state].T, dip, seg),
         _pack(w_x_f[dt_rank + d_state:].T, dip, seg)],
        axis=1).astype(cdt)
    w_dt_t = _pack(w_dt.T, dtp, dip).astype(cdt)
    b_dt_p = _pack(b_dt[None, :], 1, dip)
    neg_a = _pack((-jnp.exp(A_log)).T * 1.4426950408889634, d_state, dip)
    d_skip = _pack(D[None, :], 1, dip)
    w_out_t = _pack(w_out.T, dip, Dp).astype(cdt)

    hid = jnp.pad(hidden_states.astype(cdt),
                  ((0, 0), (0, Lp - L), (0, Dp - Dm)))

    body = functools.partial(_mamba_body, dip=dip, d_state=d_state,
                             d_conv=d_conv, dtp=dtp, seg=seg, tl=tl,
                             cpad=cpad)

    flops = 2 * B * L * (Dm * 2 * d_inner + d_inner * (dtp + 2 * seg)
                         + dtp * d_inner + d_conv * d_inner
                         + 3 * d_inner * d_state + d_inner * Dm)
    transcend = B * L * d_inner * (d_state + 4)
    bytes_acc = (int(hid.size) * 2 + B * Lp * Dp * 2
                 + (int(w_in_t.size) + int(w_x_t.size) + int(w_dt_t.size)
                    + int(w_out_t.size)) * 2
                 + (int(w_conv_t.size) + int(b_conv_p.size) + int(b_dt_p.size)
                    + int(neg_a.size) + int(d_skip.size)) * 4)

    def wspec(shape, mode):
        if mode is None:
            return pl.BlockSpec(shape, lambda b, t: (0, 0))
        return pl.BlockSpec(shape, lambda b, t: (0, 0), pipeline_mode=mode)

    def run(mode):
        grid_spec = pltpu.PrefetchScalarGridSpec(
            num_scalar_prefetch=0,
            grid=(B, n_lt),
            in_specs=[
                pl.BlockSpec((1, tl, Dp), lambda b, t: (b, t, 0)),
                wspec(w_in_t.shape, mode),
                wspec(w_conv_t.shape, mode),
                wspec(b_conv_p.shape, mode),
                wspec(w_x_t.shape, mode),
                wspec(w_dt_t.shape, mode),
                wspec(b_dt_p.shape, mode),
                wspec(neg_a.shape, mode),
                wspec(d_skip.shape, mode),
                wspec(w_out_t.shape, mode),
            ],
            out_specs=pl.BlockSpec((1, tl, Dp), lambda b, t: (b, t, 0)),
            scratch_shapes=[
                pltpu.VMEM((d_state, dip), _F32),
                pltpu.VMEM((d_conv - 1, dip), _F32),
                pltpu.VMEM((cpad + tl, dip), _F32),
                pltpu.VMEM((tl * 8, dip), _F32),
                pltpu.VMEM((tl * 8, dip), _F32),
                pltpu.VMEM((tl, d_state), _F32),
                pltpu.VMEM((tl, d_state), _F32),
                pltpu.VMEM((tl, dip), _F32),
                pltpu.VMEM((tl, dip), _F32),
                pltpu.VMEM((tl * 8, dip), _F32),
                pltpu.VMEM((tl, dip), _F32),
            ],
        )
        return pl.pallas_call(
            body,
            out_shape=jax.ShapeDtypeStruct((B, Lp, Dp), cdt),
            grid_spec=grid_spec,
            compiler_params=pltpu.CompilerParams(
                dimension_semantics=("parallel", "arbitrary"),
                vmem_limit_bytes=int(64 * 1024 * 1024 * 0.9)),
            cost_estimate=pl.CostEstimate(
                flops=int(flops), transcendentals=int(transcend),
                bytes_accessed=int(bytes_acc)),
        )(hid, w_in_t, w_conv_t, b_conv_p, w_x_t, w_dt_t, b_dt_p, neg_a,
          d_skip, w_out_t)

    try:
        out = run(pl.Buffered(1))
    except Exception:
        out = run(None)
    return out[:, :L, :Dm].astype(_F32)


# speedup vs baseline: 1.1445x; 1.1445x over previous
import functools
import math

import jax
import jax.numpy as jnp
from jax.experimental import pallas as pl
from jax.experimental.pallas import tpu as pltpu

_F32 = jnp.float32


def _ceil_to(v, m):
    return (v + m - 1) // m * m


def _mamba_body(hid_ref, w_in_ref, w_conv_ref, b_conv_ref, w_x_ref,
                w_dt_ref, b_dt_ref, neg_a_ref, d_ref, w_out_ref,
                out_ref,
                state_ref, ctail_ref, xs_ref, dl_ref, dx_ref,
                bb_ref, cc_ref, xa_ref, z_ref, y_ref, ya_ref,
                *, dip, d_state, d_conv, dtp, seg, tl, cpad):
    t_idx = pl.program_id(1)

    @pl.when(t_idx == 0)
    def _zero_carry():
        state_ref[...] = jnp.zeros_like(state_ref)
        ctail_ref[...] = jnp.zeros_like(ctail_ref)

    h = hid_ref[0]
    xz = jnp.dot(h, w_in_ref[...], preferred_element_type=_F32)
    x = xz[:, :dip]
    z_ref[...] = xz[:, dip:]

    lo = cpad - (d_conv - 1)
    xs_ref[lo:cpad, :] = ctail_ref[...]
    xs_ref[cpad:cpad + tl, :] = x
    ctail_ref[...] = xs_ref[cpad + tl - (d_conv - 1):cpad + tl, :]
    acc = b_conv_ref[...]
    for k in range(d_conv):
        acc = acc + xs_ref[lo + k:lo + k + tl, :] * w_conv_ref[k:k + 1, :]
    x_act = acc * jax.nn.sigmoid(acc)
    xa_ref[...] = x_act

    xd = jnp.dot(x_act.astype(w_x_ref.dtype), w_x_ref[...],
                 preferred_element_type=_F32)
    bb_ref[...] = xd[:, dtp:dtp + d_state]
    cc_ref[...] = xd[:, dtp + seg:dtp + seg + d_state]
    dt = jnp.dot(xd[:, :dtp].astype(w_dt_ref.dtype), w_dt_ref[...],
                 preferred_element_type=_F32)
    delta = jax.nn.softplus(dt + b_dt_ref[...])
    dl_ref[...] = delta
    dx_ref[...] = delta * x_act

    r_i = jax.lax.broadcasted_iota(jnp.int32, (d_state, d_state), 0)
    c_i = jax.lax.broadcasted_iota(jnp.int32, (d_state, d_state), 1)
    diag = (r_i == c_i).astype(_F32)

    hw = dip // 2

    def half_step(lo, na):
        def stepf(l, st):
            dl16 = dl_ref[pl.ds(l, 1), lo:lo + hw]
            dx16 = dx_ref[pl.ds(l, 1), lo:lo + hw]
            b_col = jnp.sum(diag * bb_ref[pl.ds(l, 1), :], axis=1,
                            keepdims=True)
            c_col = jnp.sum(diag * cc_ref[pl.ds(l, 1), :], axis=1,
                            keepdims=True)
            dA = jnp.exp2(dl16 * na)
            st = st * dA + b_col * dx16
            yp = st * c_col
            y_ref[pl.ds(l * 8, 8), lo:lo + hw] = (
                yp[0:8, :] + yp[8:16, :]).astype(y_ref.dtype)
            return st
        return stepf

    for lo in (0, hw):
        na = neg_a_ref[:, lo:lo + hw]
        st = jax.lax.fori_loop(0, tl, half_step(lo, na),
                               state_ref[:, lo:lo + hw], unroll=16)
        state_ref[:, lo:lo + hw] = st

    s_i = jax.lax.broadcasted_iota(jnp.int32, (8, 64), 0)
    l_i = jax.lax.broadcasted_iota(jnp.int32, (8, 64), 1)
    o8 = (l_i // 8 == s_i).astype(y_ref.dtype)
    xa = xa_ref[...]
    z = z_ref[...]
    gate = z * jax.nn.sigmoid(z)
    for g in range(tl // 8):
        yg = jnp.dot(o8, y_ref[g * 64:(g + 1) * 64, :],
                     preferred_element_type=_F32)
        r = slice(g * 8, (g + 1) * 8)
        ya_ref[r, :] = (yg + xa[r, :] * d_ref[...]) * gate[r, :]
    out = jnp.dot(ya_ref[...].astype(w_out_ref.dtype), w_out_ref[...],
                  preferred_element_type=_F32)
    out_ref[0] = out.astype(out_ref.dtype)


def _pack(a, rows, cols):
    a = a.astype(_F32)
    return jnp.pad(a, ((0, rows - a.shape[0]), (0, cols - a.shape[1])))


def kernel(hidden_states, w_in, w_conv, b_conv, w_x, w_dt, b_dt, A_log, D,
           w_out):
    d_state, d_conv, dt_rank = 16, 4, 48
    tl = 512
    cdt = jnp.bfloat16

    B, L, Dm = hidden_states.shape
    d_inner = w_in.shape[0] // 2

    Dp = _ceil_to(Dm, 128)
    dip = _ceil_to(d_inner, 128)
    tl = _ceil_to(max(16, min(tl, L)), 16)
    Lp = _ceil_to(L, tl)
    n_lt = Lp // tl
    dtp = _ceil_to(dt_rank, 128)
    seg = _ceil_to(d_state, 128)
    cpad = max(8, _ceil_to(d_conv - 1, 8))

    w_in_t = jnp.concatenate(
        [_pack(w_in[:d_inner].T, Dp, dip), _pack(w_in[d_inner:].T, Dp, dip)],
        axis=1).astype(cdt)
    w_conv_t = _pack(w_conv[:, 0, :].T, d_conv, dip)
    b_conv_p = _pack(b_conv[None, :], 1, dip)
    w_x_f = w_x.astype(_F32)
    w_x_t = jnp.concatenate(
        [_pack(w_x_f[:dt_rank].T, dip, dtp),
         _pack(w_x_f[dt_rank:dt_rank + d_state].T, dip, seg),
         _pack(w_x_f[dt_rank + d_state:].T, dip, seg)],
        axis=1).astype(cdt)
    w_dt_t = _pack(w_dt.T, dtp, dip).astype(cdt)
    b_dt_p = _pack(b_dt[None, :], 1, dip)
    neg_a = _pack((-jnp.exp(A_log)).T * 1.4426950408889634, d_state, dip)
    d_skip = _pack(D[None, :], 1, dip)
    w_out_t = _pack(w_out.T, dip, Dp).astype(cdt)

    hid = jnp.pad(hidden_states.astype(cdt),
                  ((0, 0), (0, Lp - L), (0, Dp - Dm)))

    body = functools.partial(_mamba_body, dip=dip, d_state=d_state,
                             d_conv=d_conv, dtp=dtp, seg=seg, tl=tl,
                             cpad=cpad)

    flops = 2 * B * L * (Dm * 2 * d_inner + d_inner * (dtp + 2 * seg)
                         + dtp * d_inner + d_conv * d_inner
                         + 3 * d_inner * d_state + d_inner * Dm)
    transcend = B * L * d_inner * (d_state + 4)
    bytes_acc = (int(hid.size) * 2 + B * Lp * Dp * 2
                 + (int(w_in_t.size) + int(w_x_t.size) + int(w_dt_t.size)
                    + int(w_out_t.size)) * 2
                 + (int(w_conv_t.size) + int(b_conv_p.size) + int(b_dt_p.size)
                    + int(neg_a.size) + int(d_skip.size)) * 4)

    def wspec(shape, mode):
        if mode is None:
            return pl.BlockSpec(shape, lambda b, t: (0, 0))
        return pl.BlockSpec(shape, lambda b, t: (0, 0), pipeline_mode=mode)

    def run(mode):
        grid_spec = pltpu.PrefetchScalarGridSpec(
            num_scalar_prefetch=0,
            grid=(B, n_lt),
            in_specs=[
                pl.BlockSpec((1, tl, Dp), lambda b, t: (b, t, 0)),
                wspec(w_in_t.shape, mode),
                wspec(w_conv_t.shape, mode),
                wspec(b_conv_p.shape, mode),
                wspec(w_x_t.shape, mode),
                wspec(w_dt_t.shape, mode),
                wspec(b_dt_p.shape, mode),
                wspec(neg_a.shape, mode),
                wspec(d_skip.shape, mode),
                wspec(w_out_t.shape, mode),
            ],
            out_specs=pl.BlockSpec((1, tl, Dp), lambda b, t: (b, t, 0)),
            scratch_shapes=[
                pltpu.VMEM((d_state, dip), _F32),
                pltpu.VMEM((d_conv - 1, dip), _F32),
                pltpu.VMEM((cpad + tl, dip), _F32),
                pltpu.VMEM((tl, dip), _F32),
                pltpu.VMEM((tl, dip), _F32),
                pltpu.VMEM((tl, d_state), _F32),
                pltpu.VMEM((tl, d_state), _F32),
                pltpu.VMEM((tl, dip), _F32),
                pltpu.VMEM((tl, dip), _F32),
                pltpu.VMEM((tl * 8, dip), _F32),
                pltpu.VMEM((tl, dip), _F32),
            ],
        )
        return pl.pallas_call(
            body,
            out_shape=jax.ShapeDtypeStruct((B, Lp, Dp), cdt),
            grid_spec=grid_spec,
            compiler_params=pltpu.CompilerParams(
                dimension_semantics=("parallel", "arbitrary"),
                vmem_limit_bytes=int(64 * 1024 * 1024 * 0.9)),
            cost_estimate=pl.CostEstimate(
                flops=int(flops), transcendentals=int(transcend),
                bytes_accessed=int(bytes_acc)),
        )(hid, w_in_t, w_conv_t, b_conv_p, w_x_t, w_dt_t, b_dt_p, neg_a,
          d_skip, w_out_t)

    try:
        out = run(pl.Buffered(1))
    except Exception:
        out = run(None)
    return out[:, :L, :Dm].astype(_F32)


# speedup vs baseline: 1.1983x; 1.0470x over previous
import functools
import math

import jax
import jax.numpy as jnp
from jax.experimental import pallas as pl
from jax.experimental.pallas import tpu as pltpu

_F32 = jnp.float32


def _ceil_to(v, m):
    return (v + m - 1) // m * m


def _mamba_body(hid_ref, w_in_ref, w_conv_ref, b_conv_ref, w_x_ref,
                w_dt_ref, b_dt_ref, neg_a_ref, d_ref, w_out_ref,
                out_ref,
                state_ref, ctail_ref, xs_ref, dl_ref, dx_ref,
                bb_ref, cc_ref, xa_ref, z_ref, y_ref, ya_ref,
                *, dip, d_state, d_conv, dtp, seg, tl, cpad):
    t_idx = pl.program_id(1)

    @pl.when(t_idx == 0)
    def _zero_carry():
        state_ref[...] = jnp.zeros_like(state_ref)
        ctail_ref[...] = jnp.zeros_like(ctail_ref)

    h = hid_ref[0]
    xz = jnp.dot(h, w_in_ref[...], preferred_element_type=_F32)
    x = xz[:, :dip]
    z_ref[...] = xz[:, dip:]

    lo = cpad - (d_conv - 1)
    xs_ref[lo:cpad, :] = ctail_ref[...]
    xs_ref[cpad:cpad + tl, :] = x
    ctail_ref[...] = xs_ref[cpad + tl - (d_conv - 1):cpad + tl, :]
    acc = b_conv_ref[...]
    for k in range(d_conv):
        acc = acc + xs_ref[lo + k:lo + k + tl, :] * w_conv_ref[k:k + 1, :]
    x_act = acc * jax.nn.sigmoid(acc)
    xa_ref[...] = x_act

    xd = jnp.dot(x_act.astype(w_x_ref.dtype), w_x_ref[...],
                 preferred_element_type=_F32)
    bb_ref[...] = xd[:, dtp:dtp + d_state]
    cc_ref[...] = xd[:, dtp + seg:dtp + seg + d_state]
    dt = jnp.dot(xd[:, :dtp].astype(w_dt_ref.dtype), w_dt_ref[...],
                 preferred_element_type=_F32)
    delta = jax.nn.softplus(dt + b_dt_ref[...])
    dl_ref[...] = delta
    dx_ref[...] = delta * x_act

    r_i = jax.lax.broadcasted_iota(jnp.int32, (d_state, d_state), 0)
    c_i = jax.lax.broadcasted_iota(jnp.int32, (d_state, d_state), 1)
    diag = (r_i == c_i).astype(_F32)

    hw = dip // 2

    def half_step(lo, na):
        def stepf(l, st):
            dl16 = dl_ref[pl.ds(l, 1), lo:lo + hw]
            dx16 = dx_ref[pl.ds(l, 1), lo:lo + hw]
            b_col = jnp.sum(diag * bb_ref[pl.ds(l, 1), :], axis=1,
                            keepdims=True)
            c_col = jnp.sum(diag * cc_ref[pl.ds(l, 1), :], axis=1,
                            keepdims=True)
            dA = jnp.exp2(dl16 * na)
            st = st * dA + b_col * dx16
            yp = st * c_col
            y_ref[pl.ds(l * 8, 8), lo:lo + hw] = (
                yp[0:8, :] + yp[8:16, :]).astype(y_ref.dtype)
            return st
        return stepf

    for lo in (0, hw):
        na = neg_a_ref[:, lo:lo + hw]
        st = jax.lax.fori_loop(0, tl, half_step(lo, na),
                               state_ref[:, lo:lo + hw], unroll=32)
        state_ref[:, lo:lo + hw] = st

    s_i = jax.lax.broadcasted_iota(jnp.int32, (8, 64), 0)
    l_i = jax.lax.broadcasted_iota(jnp.int32, (8, 64), 1)
    o8 = (l_i // 8 == s_i).astype(y_ref.dtype)
    xa = xa_ref[...]
    z = z_ref[...]
    gate = z * jax.nn.sigmoid(z)
    for g in range(tl // 8):
        yg = jnp.dot(o8, y_ref[g * 64:(g + 1) * 64, :],
                     preferred_element_type=_F32)
        r = slice(g * 8, (g + 1) * 8)
        ya_ref[r, :] = (yg + xa[r, :] * d_ref[...]) * gate[r, :]
    out = jnp.dot(ya_ref[...].astype(w_out_ref.dtype), w_out_ref[...],
                  preferred_element_type=_F32)
    out_ref[0] = out.astype(out_ref.dtype)


def _pack(a, rows, cols):
    a = a.astype(_F32)
    return jnp.pad(a, ((0, rows - a.shape[0]), (0, cols - a.shape[1])))


def kernel(hidden_states, w_in, w_conv, b_conv, w_x, w_dt, b_dt, A_log, D,
           w_out):
    d_state, d_conv, dt_rank = 16, 4, 48
    tl = 512
    cdt = jnp.bfloat16

    B, L, Dm = hidden_states.shape
    d_inner = w_in.shape[0] // 2

    Dp = _ceil_to(Dm, 128)
    dip = _ceil_to(d_inner, 128)
    tl = _ceil_to(max(16, min(tl, L)), 16)
    Lp = _ceil_to(L, tl)
    n_lt = Lp // tl
    dtp = _ceil_to(dt_rank, 128)
    seg = _ceil_to(d_state, 128)
    cpad = max(8, _ceil_to(d_conv - 1, 8))

    w_in_t = jnp.concatenate(
        [_pack(w_in[:d_inner].T, Dp, dip), _pack(w_in[d_inner:].T, Dp, dip)],
        axis=1).astype(cdt)
    w_conv_t = _pack(w_conv[:, 0, :].T, d_conv, dip)
    b_conv_p = _pack(b_conv[None, :], 1, dip)
    w_x_f = w_x.astype(_F32)
    w_x_t = jnp.concatenate(
        [_pack(w_x_f[:dt_rank].T, dip, dtp),
         _pack(w_x_f[dt_rank:dt_rank + d_state].T, dip, seg),
         _pack(w_x_f[dt_rank + d_state:].T, dip, seg)],
        axis=1).astype(cdt)
    w_dt_t = _pack(w_dt.T, dtp, dip).astype(cdt)
    b_dt_p = _pack(b_dt[None, :], 1, dip)
    neg_a = _pack((-jnp.exp(A_log)).T * 1.4426950408889634, d_state, dip)
    d_skip = _pack(D[None, :], 1, dip)
    w_out_t = _pack(w_out.T, dip, Dp).astype(cdt)

    hid = jnp.pad(hidden_states.astype(cdt),
                  ((0, 0), (0, Lp - L), (0, Dp - Dm)))

    body = functools.partial(_mamba_body, dip=dip, d_state=d_state,
                             d_conv=d_conv, dtp=dtp, seg=seg, tl=tl,
                             cpad=cpad)

    flops = 2 * B * L * (Dm * 2 * d_inner + d_inner * (dtp + 2 * seg)
                         + dtp * d_inner + d_conv * d_inner
                         + 3 * d_inner * d_state + d_inner * Dm)
    transcend = B * L * d_inner * (d_state + 4)
    bytes_acc = (int(hid.size) * 2 + B * Lp * Dp * 2
                 + (int(w_in_t.size) + int(w_x_t.size) + int(w_dt_t.size)
                    + int(w_out_t.size)) * 2
                 + (int(w_conv_t.size) + int(b_conv_p.size) + int(b_dt_p.size)
                    + int(neg_a.size) + int(d_skip.size)) * 4)

    def wspec(shape, mode):
        if mode is None:
            return pl.BlockSpec(shape, lambda b, t: (0, 0))
        return pl.BlockSpec(shape, lambda b, t: (0, 0), pipeline_mode=mode)

    def run(mode):
        grid_spec = pltpu.PrefetchScalarGridSpec(
            num_scalar_prefetch=0,
            grid=(B, n_lt),
            in_specs=[
                pl.BlockSpec((1, tl, Dp), lambda b, t: (b, t, 0)),
                wspec(w_in_t.shape, mode),
                wspec(w_conv_t.shape, mode),
                wspec(b_conv_p.shape, mode),
                wspec(w_x_t.shape, mode),
                wspec(w_dt_t.shape, mode),
                wspec(b_dt_p.shape, mode),
                wspec(neg_a.shape, mode),
                wspec(d_skip.shape, mode),
                wspec(w_out_t.shape, mode),
            ],
            out_specs=pl.BlockSpec((1, tl, Dp), lambda b, t: (b, t, 0)),
            scratch_shapes=[
                pltpu.VMEM((d_state, dip), _F32),
                pltpu.VMEM((d_conv - 1, dip), _F32),
                pltpu.VMEM((cpad + tl, dip), _F32),
                pltpu.VMEM((tl, dip), _F32),
                pltpu.VMEM((tl, dip), _F32),
                pltpu.VMEM((tl, d_state), _F32),
                pltpu.VMEM((tl, d_state), _F32),
                pltpu.VMEM((tl, dip), _F32),
                pltpu.VMEM((tl, dip), _F32),
                pltpu.VMEM((tl * 8, dip), _F32),
                pltpu.VMEM((tl, dip), _F32),
            ],
        )
        return pl.pallas_call(
            body,
            out_shape=jax.ShapeDtypeStruct((B, Lp, Dp), cdt),
            grid_spec=grid_spec,
            compiler_params=pltpu.CompilerParams(
                dimension_semantics=("parallel", "arbitrary"),
                vmem_limit_bytes=int(64 * 1024 * 1024 * 0.9)),
            cost_estimate=pl.CostEstimate(
                flops=int(flops), transcendentals=int(transcend),
                bytes_accessed=int(bytes_acc)),
        )(hid, w_in_t, w_conv_t, b_conv_p, w_x_t, w_dt_t, b_dt_p, neg_a,
          d_skip, w_out_t)

    try:
        out = run(pl.Buffered(1))
    except Exception:
        out = run(None)
    return out[:, :L, :Dm].astype(_F32)


# speedup vs baseline: 1.2299x; 1.0263x over previous
import functools
import math

import jax
import jax.numpy as jnp
from jax.experimental import pallas as pl
from jax.experimental.pallas import tpu as pltpu

_F32 = jnp.float32


def _ceil_to(v, m):
    return (v + m - 1) // m * m


def _mamba_body(hid_ref, w_in_ref, w_conv_ref, b_conv_ref, w_x_ref,
                w_dt_ref, b_dt_ref, neg_a_ref, d_ref, w_out_ref,
                out_ref,
                state_ref, ctail_ref, xs_ref, dl_ref, dx_ref,
                bb_ref, cc_ref, xa_ref, z_ref, y_ref, ya_ref,
                *, dip, d_state, d_conv, dtp, seg, tl, cpad):
    t_idx = pl.program_id(1)

    @pl.when(t_idx == 0)
    def _zero_carry():
        state_ref[...] = jnp.zeros_like(state_ref)
        ctail_ref[...] = jnp.zeros_like(ctail_ref)

    h = hid_ref[0]
    xz = jnp.dot(h, w_in_ref[...], preferred_element_type=_F32)
    x = xz[:, :dip]
    z_ref[...] = xz[:, dip:]

    lo = cpad - (d_conv - 1)
    xs_ref[lo:cpad, :] = ctail_ref[...]
    xs_ref[cpad:cpad + tl, :] = x
    ctail_ref[...] = xs_ref[cpad + tl - (d_conv - 1):cpad + tl, :]
    acc = b_conv_ref[...]
    for k in range(d_conv):
        acc = acc + xs_ref[lo + k:lo + k + tl, :] * w_conv_ref[k:k + 1, :]
    x_act = acc * jax.nn.sigmoid(acc)
    xa_ref[...] = x_act

    xd = jnp.dot(x_act.astype(w_x_ref.dtype), w_x_ref[...],
                 preferred_element_type=_F32)
    bb_ref[...] = xd[:, dtp:dtp + d_state]
    cc_ref[...] = xd[:, dtp + seg:dtp + seg + d_state]
    dt = jnp.dot(xd[:, :dtp].astype(w_dt_ref.dtype), w_dt_ref[...],
                 preferred_element_type=_F32)
    delta = jax.nn.softplus(dt + b_dt_ref[...])
    dl_ref[...] = delta
    dx_ref[...] = delta * x_act

    r_i = jax.lax.broadcasted_iota(jnp.int32, (d_state, d_state), 0)
    c_i = jax.lax.broadcasted_iota(jnp.int32, (d_state, d_state), 1)
    diag = (r_i == c_i).astype(_F32)

    hw = dip // 2

    def half_step(lo, na):
        def stepf(l, st):
            dl16 = dl_ref[pl.ds(l, 1), lo:lo + hw]
            dx16 = dx_ref[pl.ds(l, 1), lo:lo + hw]
            b_col = jnp.sum(diag * bb_ref[pl.ds(l, 1), :], axis=1,
                            keepdims=True)
            c_col = jnp.sum(diag * cc_ref[pl.ds(l, 1), :], axis=1,
                            keepdims=True)
            dA = jnp.exp2(dl16 * na)
            st = st * dA + b_col * dx16
            yp = st * c_col
            y_ref[pl.ds(l * 8, 8), lo:lo + hw] = (
                yp[0:8, :] + yp[8:16, :]).astype(y_ref.dtype)
            return st
        return stepf

    for lo in (0, hw):
        na = neg_a_ref[:, lo:lo + hw]
        st = jax.lax.fori_loop(0, tl, half_step(lo, na),
                               state_ref[:, lo:lo + hw], unroll=64)
        state_ref[:, lo:lo + hw] = st

    s_i = jax.lax.broadcasted_iota(jnp.int32, (8, 64), 0)
    l_i = jax.lax.broadcasted_iota(jnp.int32, (8, 64), 1)
    o8 = (l_i // 8 == s_i).astype(y_ref.dtype)
    xa = xa_ref[...]
    z = z_ref[...]
    gate = z * jax.nn.sigmoid(z)
    for g in range(tl // 8):
        yg = jnp.dot(o8, y_ref[g * 64:(g + 1) * 64, :],
                     preferred_element_type=_F32)
        r = slice(g * 8, (g + 1) * 8)
        ya_ref[r, :] = (yg + xa[r, :] * d_ref[...]) * gate[r, :]
    out = jnp.dot(ya_ref[...].astype(w_out_ref.dtype), w_out_ref[...],
                  preferred_element_type=_F32)
    out_ref[0] = out.astype(out_ref.dtype)


def _pack(a, rows, cols):
    a = a.astype(_F32)
    return jnp.pad(a, ((0, rows - a.shape[0]), (0, cols - a.shape[1])))


def kernel(hidden_states, w_in, w_conv, b_conv, w_x, w_dt, b_dt, A_log, D,
           w_out):
    d_state, d_conv, dt_rank = 16, 4, 48
    tl = 512
    cdt = jnp.bfloat16

    B, L, Dm = hidden_states.shape
    d_inner = w_in.shape[0] // 2

    Dp = _ceil_to(Dm, 128)
    dip = _ceil_to(d_inner, 128)
    tl = _ceil_to(max(16, min(tl, L)), 16)
    Lp = _ceil_to(L, tl)
    n_lt = Lp // tl
    dtp = _ceil_to(dt_rank, 128)
    seg = _ceil_to(d_state, 128)
    cpad = max(8, _ceil_to(d_conv - 1, 8))

    w_in_t = jnp.concatenate(
        [_pack(w_in[:d_inner].T, Dp, dip), _pack(w_in[d_inner:].T, Dp, dip)],
        axis=1).astype(cdt)
    w_conv_t = _pack(w_conv[:, 0, :].T, d_conv, dip)
    b_conv_p = _pack(b_conv[None, :], 1, dip)
    w_x_f = w_x.astype(_F32)
    w_x_t = jnp.concatenate(
        [_pack(w_x_f[:dt_rank].T, dip, dtp),
         _pack(w_x_f[dt_rank:dt_rank + d_state].T, dip, seg),
         _pack(w_x_f[dt_rank + d_state:].T, dip, seg)],
        axis=1).astype(cdt)
    w_dt_t = _pack(w_dt.T, dtp, dip).astype(cdt)
    b_dt_p = _pack(b_dt[None, :], 1, dip)
    neg_a = _pack((-jnp.exp(A_log)).T * 1.4426950408889634, d_state, dip)
    d_skip = _pack(D[None, :], 1, dip)
    w_out_t = _pack(w_out.T, dip, Dp).astype(cdt)

    hid = jnp.pad(hidden_states.astype(cdt),
                  ((0, 0), (0, Lp - L), (0, Dp - Dm)))

    body = functools.partial(_mamba_body, dip=dip, d_state=d_state,
                             d_conv=d_conv, dtp=dtp, seg=seg, tl=tl,
                             cpad=cpad)

    flops = 2 * B * L * (Dm * 2 * d_inner + d_inner * (dtp + 2 * seg)
                         + dtp * d_inner + d_conv * d_inner
                         + 3 * d_inner * d_state + d_inner * Dm)
    transcend = B * L * d_inner * (d_state + 4)
    bytes_acc = (int(hid.size) * 2 + B * Lp * Dp * 2
                 + (int(w_in_t.size) + int(w_x_t.size) + int(w_dt_t.size)
                    + int(w_out_t.size)) * 2
                 + (int(w_conv_t.size) + int(b_conv_p.size) + int(b_dt_p.size)
                    + int(neg_a.size) + int(d_skip.size)) * 4)

    def wspec(shape, mode):
        if mode is None:
            return pl.BlockSpec(shape, lambda b, t: (0, 0))
        return pl.BlockSpec(shape, lambda b, t: (0, 0), pipeline_mode=mode)

    def run(mode):
        grid_spec = pltpu.PrefetchScalarGridSpec(
            num_scalar_prefetch=0,
            grid=(B, n_lt),
            in_specs=[
                pl.BlockSpec((1, tl, Dp), lambda b, t: (b, t, 0)),
                wspec(w_in_t.shape, mode),
                wspec(w_conv_t.shape, mode),
                wspec(b_conv_p.shape, mode),
                wspec(w_x_t.shape, mode),
                wspec(w_dt_t.shape, mode),
                wspec(b_dt_p.shape, mode),
                wspec(neg_a.shape, mode),
                wspec(d_skip.shape, mode),
                wspec(w_out_t.shape, mode),
            ],
            out_specs=pl.BlockSpec((1, tl, Dp), lambda b, t: (b, t, 0)),
            scratch_shapes=[
                pltpu.VMEM((d_state, dip), _F32),
                pltpu.VMEM((d_conv - 1, dip), _F32),
                pltpu.VMEM((cpad + tl, dip), _F32),
                pltpu.VMEM((tl, dip), _F32),
                pltpu.VMEM((tl, dip), _F32),
                pltpu.VMEM((tl, d_state), _F32),
                pltpu.VMEM((tl, d_state), _F32),
                pltpu.VMEM((tl, dip), _F32),
                pltpu.VMEM((tl, dip), _F32),
                pltpu.VMEM((tl * 8, dip), _F32),
                pltpu.VMEM((tl, dip), _F32),
            ],
        )
        return pl.pallas_call(
            body,
            out_shape=jax.ShapeDtypeStruct((B, Lp, Dp), cdt),
            grid_spec=grid_spec,
            compiler_params=pltpu.CompilerParams(
                dimension_semantics=("parallel", "arbitrary"),
                vmem_limit_bytes=int(64 * 1024 * 1024 * 0.9)),
            cost_estimate=pl.CostEstimate(
                flops=int(flops), transcendentals=int(transcend),
                bytes_accessed=int(bytes_acc)),
        )(hid, w_in_t, w_conv_t, b_conv_p, w_x_t, w_dt_t, b_dt_p, neg_a,
          d_skip, w_out_t)

    try:
        out = run(pl.Buffered(1))
    except Exception:
        out = run(None)
    return out[:, :L, :Dm].astype(_F32)


# speedup vs baseline: 1.2549x; 1.0203x over previous
import functools
import math

import jax
import jax.numpy as jnp
from jax.experimental import pallas as pl
from jax.experimental.pallas import tpu as pltpu

_F32 = jnp.float32


def _ceil_to(v, m):
    return (v + m - 1) // m * m


def _mamba_body(hid_ref, w_in_ref, w_conv_ref, b_conv_ref, w_x_ref,
                w_dt_ref, b_dt_ref, neg_a_ref, d_ref, w_out_ref,
                out_ref,
                state_ref, ctail_ref, xs_ref, dl_ref, dx_ref,
                bb_ref, cc_ref, xa_ref, z_ref, y_ref, ya_ref,
                *, dip, d_state, d_conv, dtp, seg, tl, cpad):
    t_idx = pl.program_id(1)

    @pl.when(t_idx == 0)
    def _zero_carry():
        state_ref[...] = jnp.zeros_like(state_ref)
        ctail_ref[...] = jnp.zeros_like(ctail_ref)

    h = hid_ref[0]
    xz = jnp.dot(h, w_in_ref[...], preferred_element_type=_F32)
    x = xz[:, :dip]
    z_ref[...] = xz[:, dip:]

    lo = cpad - (d_conv - 1)
    xs_ref[lo:cpad, :] = ctail_ref[...]
    xs_ref[cpad:cpad + tl, :] = x
    ctail_ref[...] = xs_ref[cpad + tl - (d_conv - 1):cpad + tl, :]
    acc = b_conv_ref[...]
    for k in range(d_conv):
        acc = acc + xs_ref[lo + k:lo + k + tl, :] * w_conv_ref[k:k + 1, :]
    x_act = acc * jax.nn.sigmoid(acc)
    xa_ref[...] = x_act

    xd = jnp.dot(x_act.astype(w_x_ref.dtype), w_x_ref[...],
                 preferred_element_type=_F32)
    bb_ref[...] = xd[:, dtp:dtp + d_state]
    cc_ref[...] = xd[:, dtp + seg:dtp + seg + d_state]
    dt = jnp.dot(xd[:, :dtp].astype(w_dt_ref.dtype), w_dt_ref[...],
                 preferred_element_type=_F32)
    delta = jax.nn.softplus(dt + b_dt_ref[...])
    dl_ref[...] = delta
    dx_ref[...] = delta * x_act

    r_i = jax.lax.broadcasted_iota(jnp.int32, (d_state, d_state), 0)
    c_i = jax.lax.broadcasted_iota(jnp.int32, (d_state, d_state), 1)
    diag = (r_i == c_i).astype(_F32)

    hw = dip // 2

    def half_step(lo, na):
        def stepf(l, st):
            dl16 = dl_ref[pl.ds(l, 1), lo:lo + hw]
            dx16 = dx_ref[pl.ds(l, 1), lo:lo + hw]
            b_col = jnp.sum(diag * bb_ref[pl.ds(l, 1), :], axis=1,
                            keepdims=True)
            c_col = jnp.sum(diag * cc_ref[pl.ds(l, 1), :], axis=1,
                            keepdims=True)
            dA = jnp.exp2(dl16 * na)
            st = st * dA + b_col * dx16
            yp = st * c_col
            y_ref[pl.ds(l * 8, 8), lo:lo + hw] = (
                yp[0:8, :] + yp[8:16, :]).astype(y_ref.dtype)
            return st
        return stepf

    for lo in (0, hw):
        na = neg_a_ref[:, lo:lo + hw]
        st = jax.lax.fori_loop(0, tl, half_step(lo, na),
                               state_ref[:, lo:lo + hw], unroll=128)
        state_ref[:, lo:lo + hw] = st

    s_i = jax.lax.broadcasted_iota(jnp.int32, (8, 64), 0)
    l_i = jax.lax.broadcasted_iota(jnp.int32, (8, 64), 1)
    o8 = (l_i // 8 == s_i).astype(y_ref.dtype)
    xa = xa_ref[...]
    z = z_ref[...]
    gate = z * jax.nn.sigmoid(z)
    for g in range(tl // 8):
        yg = jnp.dot(o8, y_ref[g * 64:(g + 1) * 64, :],
                     preferred_element_type=_F32)
        r = slice(g * 8, (g + 1) * 8)
        ya_ref[r, :] = (yg + xa[r, :] * d_ref[...]) * gate[r, :]
    out = jnp.dot(ya_ref[...].astype(w_out_ref.dtype), w_out_ref[...],
                  preferred_element_type=_F32)
    out_ref[0] = out.astype(out_ref.dtype)


def _pack(a, rows, cols):
    a = a.astype(_F32)
    return jnp.pad(a, ((0, rows - a.shape[0]), (0, cols - a.shape[1])))


def kernel(hidden_states, w_in, w_conv, b_conv, w_x, w_dt, b_dt, A_log, D,
           w_out):
    d_state, d_conv, dt_rank = 16, 4, 48
    tl = 512
    cdt = jnp.bfloat16

    B, L, Dm = hidden_states.shape
    d_inner = w_in.shape[0] // 2

    Dp = _ceil_to(Dm, 128)
    dip = _ceil_to(d_inner, 128)
    tl = _ceil_to(max(16, min(tl, L)), 16)
    Lp = _ceil_to(L, tl)
    n_lt = Lp // tl
    dtp = _ceil_to(dt_rank, 128)
    seg = _ceil_to(d_state, 128)
    cpad = max(8, _ceil_to(d_conv - 1, 8))

    w_in_t = jnp.concatenate(
        [_pack(w_in[:d_inner].T, Dp, dip), _pack(w_in[d_inner:].T, Dp, dip)],
        axis=1).astype(cdt)
    w_conv_t = _pack(w_conv[:, 0, :].T, d_conv, dip)
    b_conv_p = _pack(b_conv[None, :], 1, dip)
    w_x_f = w_x.astype(_F32)
    w_x_t = jnp.concatenate(
        [_pack(w_x_f[:dt_rank].T, dip, dtp),
         _pack(w_x_f[dt_rank:dt_rank + d_state].T, dip, seg),
         _pack(w_x_f[dt_rank + d_state:].T, dip, seg)],
        axis=1).astype(cdt)
    w_dt_t = _pack(w_dt.T, dtp, dip).astype(cdt)
    b_dt_p = _pack(b_dt[None, :], 1, dip)
    neg_a = _pack((-jnp.exp(A_log)).T * 1.4426950408889634, d_state, dip)
    d_skip = _pack(D[None, :], 1, dip)
    w_out_t = _pack(w_out.T, dip, Dp).astype(cdt)

    hid = jnp.pad(hidden_states.astype(cdt),
                  ((0, 0), (0, Lp - L), (0, Dp - Dm)))

    body = functools.partial(_mamba_body, dip=dip, d_state=d_state,
                             d_conv=d_conv, dtp=dtp, seg=seg, tl=tl,
                             cpad=cpad)

    flops = 2 * B * L * (Dm * 2 * d_inner + d_inner * (dtp + 2 * seg)
                         + dtp * d_inner + d_conv * d_inner
                         + 3 * d_inner * d_state + d_inner * Dm)
    transcend = B * L * d_inner * (d_state + 4)
    bytes_acc = (int(hid.size) * 2 + B * Lp * Dp * 2
                 + (int(w_in_t.size) + int(w_x_t.size) + int(w_dt_t.size)
                    + int(w_out_t.size)) * 2
                 + (int(w_conv_t.size) + int(b_conv_p.size) + int(b_dt_p.size)
                    + int(neg_a.size) + int(d_skip.size)) * 4)

    def wspec(shape, mode):
        if mode is None:
            return pl.BlockSpec(shape, lambda b, t: (0, 0))
        return pl.BlockSpec(shape, lambda b, t: (0, 0), pipeline_mode=mode)

    def run(mode):
        grid_spec = pltpu.PrefetchScalarGridSpec(
            num_scalar_prefetch=0,
            grid=(B, n_lt),
            in_specs=[
                pl.BlockSpec((1, tl, Dp), lambda b, t: (b, t, 0)),
                wspec(w_in_t.shape, mode),
                wspec(w_conv_t.shape, mode),
                wspec(b_conv_p.shape, mode),
                wspec(w_x_t.shape, mode),
                wspec(w_dt_t.shape, mode),
                wspec(b_dt_p.shape, mode),
                wspec(neg_a.shape, mode),
                wspec(d_skip.shape, mode),
                wspec(w_out_t.shape, mode),
            ],
            out_specs=pl.BlockSpec((1, tl, Dp), lambda b, t: (b, t, 0)),
            scratch_shapes=[
                pltpu.VMEM((d_state, dip), _F32),
                pltpu.VMEM((d_conv - 1, dip), _F32),
                pltpu.VMEM((cpad + tl, dip), _F32),
                pltpu.VMEM((tl, dip), _F32),
                pltpu.VMEM((tl, dip), _F32),
                pltpu.VMEM((tl, d_state), _F32),
                pltpu.VMEM((tl, d_state), _F32),
                pltpu.VMEM((tl, dip), _F32),
                pltpu.VMEM((tl, dip), _F32),
                pltpu.VMEM((tl * 8, dip), _F32),
                pltpu.VMEM((tl, dip), _F32),
            ],
        )
        return pl.pallas_call(
            body,
            out_shape=jax.ShapeDtypeStruct((B, Lp, Dp), cdt),
            grid_spec=grid_spec,
            compiler_params=pltpu.CompilerParams(
                dimension_semantics=("parallel", "arbitrary"),
                vmem_limit_bytes=int(64 * 1024 * 1024 * 0.9)),
            cost_estimate=pl.CostEstimate(
                flops=int(flops), transcendentals=int(transcend),
                bytes_accessed=int(bytes_acc)),
        )(hid, w_in_t, w_conv_t, b_conv_p, w_x_t, w_dt_t, b_dt_p, neg_a,
          d_skip, w_out_t)

    try:
        out = run(pl.Buffered(1))
    except Exception:
        out = run(None)
    return out[:, :L, :Dm].astype(_F32)


# speedup vs baseline: 1.2704x; 1.0124x over previous
import functools
import math

import jax
import jax.numpy as jnp
from jax.experimental import pallas as pl
from jax.experimental.pallas import tpu as pltpu

_F32 = jnp.float32


def _ceil_to(v, m):
    return (v + m - 1) // m * m


def _mamba_body(hid_ref, w_in_ref, w_conv_ref, b_conv_ref, w_x_ref,
                w_dt_ref, b_dt_ref, neg_a_ref, d_ref, w_out_ref,
                out_ref,
                state_ref, ctail_ref, xs_ref, dl_ref, dx_ref,
                bb_ref, cc_ref, xa_ref, z_ref, y_ref, ya_ref,
                *, dip, d_state, d_conv, dtp, seg, tl, cpad):
    t_idx = pl.program_id(1)

    @pl.when(t_idx == 0)
    def _zero_carry():
        state_ref[...] = jnp.zeros_like(state_ref)
        ctail_ref[...] = jnp.zeros_like(ctail_ref)

    h = hid_ref[0]
    xz = jnp.dot(h, w_in_ref[...], preferred_element_type=_F32)
    x = xz[:, :dip]
    z_ref[...] = xz[:, dip:]

    lo = cpad - (d_conv - 1)
    xs_ref[lo:cpad, :] = ctail_ref[...]
    xs_ref[cpad:cpad + tl, :] = x
    ctail_ref[...] = xs_ref[cpad + tl - (d_conv - 1):cpad + tl, :]
    acc = b_conv_ref[...]
    for k in range(d_conv):
        acc = acc + xs_ref[lo + k:lo + k + tl, :] * w_conv_ref[k:k + 1, :]
    x_act = acc * jax.nn.sigmoid(acc)
    xa_ref[...] = x_act

    xd = jnp.dot(x_act.astype(w_x_ref.dtype), w_x_ref[...],
                 preferred_element_type=_F32)
    bb_ref[...] = xd[:, dtp:dtp + d_state]
    cc_ref[...] = xd[:, dtp + seg:dtp + seg + d_state]
    dt = jnp.dot(xd[:, :dtp].astype(w_dt_ref.dtype), w_dt_ref[...],
                 preferred_element_type=_F32)
    delta = jax.nn.softplus(dt + b_dt_ref[...])
    dl_ref[...] = delta
    dx_ref[...] = delta * x_act

    r_i = jax.lax.broadcasted_iota(jnp.int32, (d_state, d_state), 0)
    c_i = jax.lax.broadcasted_iota(jnp.int32, (d_state, d_state), 1)
    diag = (r_i == c_i).astype(_F32)

    hw = dip // 2

    def half_step(lo, na):
        def stepf(l, st):
            dl16 = dl_ref[pl.ds(l, 1), lo:lo + hw]
            dx16 = dx_ref[pl.ds(l, 1), lo:lo + hw]
            b_col = jnp.sum(diag * bb_ref[pl.ds(l, 1), :], axis=1,
                            keepdims=True)
            c_col = jnp.sum(diag * cc_ref[pl.ds(l, 1), :], axis=1,
                            keepdims=True)
            dA = jnp.exp2(dl16 * na)
            st = st * dA + b_col * dx16
            yp = st * c_col
            y_ref[pl.ds(l * 8, 8), lo:lo + hw] = (
                yp[0:8, :] + yp[8:16, :]).astype(y_ref.dtype)
            return st
        return stepf

    for lo in (0, hw):
        na = neg_a_ref[:, lo:lo + hw]
        st = jax.lax.fori_loop(0, tl, half_step(lo, na),
                               state_ref[:, lo:lo + hw], unroll=256)
        state_ref[:, lo:lo + hw] = st

    s_i = jax.lax.broadcasted_iota(jnp.int32, (8, 64), 0)
    l_i = jax.lax.broadcasted_iota(jnp.int32, (8, 64), 1)
    o8 = (l_i // 8 == s_i).astype(y_ref.dtype)
    xa = xa_ref[...]
    z = z_ref[...]
    gate = z * jax.nn.sigmoid(z)
    for g in range(tl // 8):
        yg = jnp.dot(o8, y_ref[g * 64:(g + 1) * 64, :],
                     preferred_element_type=_F32)
        r = slice(g * 8, (g + 1) * 8)
        ya_ref[r, :] = (yg + xa[r, :] * d_ref[...]) * gate[r, :]
    out = jnp.dot(ya_ref[...].astype(w_out_ref.dtype), w_out_ref[...],
                  preferred_element_type=_F32)
    out_ref[0] = out.astype(out_ref.dtype)


def _pack(a, rows, cols):
    a = a.astype(_F32)
    return jnp.pad(a, ((0, rows - a.shape[0]), (0, cols - a.shape[1])))


def kernel(hidden_states, w_in, w_conv, b_conv, w_x, w_dt, b_dt, A_log, D,
           w_out):
    d_state, d_conv, dt_rank = 16, 4, 48
    tl = 512
    cdt = jnp.bfloat16

    B, L, Dm = hidden_states.shape
    d_inner = w_in.shape[0] // 2

    Dp = _ceil_to(Dm, 128)
    dip = _ceil_to(d_inner, 128)
    tl = _ceil_to(max(16, min(tl, L)), 16)
    Lp = _ceil_to(L, tl)
    n_lt = Lp // tl
    dtp = _ceil_to(dt_rank, 128)
    seg = _ceil_to(d_state, 128)
    cpad = max(8, _ceil_to(d_conv - 1, 8))

    w_in_t = jnp.concatenate(
        [_pack(w_in[:d_inner].T, Dp, dip), _pack(w_in[d_inner:].T, Dp, dip)],
        axis=1).astype(cdt)
    w_conv_t = _pack(w_conv[:, 0, :].T, d_conv, dip)
    b_conv_p = _pack(b_conv[None, :], 1, dip)
    w_x_f = w_x.astype(_F32)
    w_x_t = jnp.concatenate(
        [_pack(w_x_f[:dt_rank].T, dip, dtp),
         _pack(w_x_f[dt_rank:dt_rank + d_state].T, dip, seg),
         _pack(w_x_f[dt_rank + d_state:].T, dip, seg)],
        axis=1).astype(cdt)
    w_dt_t = _pack(w_dt.T, dtp, dip).astype(cdt)
    b_dt_p = _pack(b_dt[None, :], 1, dip)
    neg_a = _pack((-jnp.exp(A_log)).T * 1.4426950408889634, d_state, dip)
    d_skip = _pack(D[None, :], 1, dip)
    w_out_t = _pack(w_out.T, dip, Dp).astype(cdt)

    hid = jnp.pad(hidden_states.astype(cdt),
                  ((0, 0), (0, Lp - L), (0, Dp - Dm)))

    body = functools.partial(_mamba_body, dip=dip, d_state=d_state,
                             d_conv=d_conv, dtp=dtp, seg=seg, tl=tl,
                             cpad=cpad)

    flops = 2 * B * L * (Dm * 2 * d_inner + d_inner * (dtp + 2 * seg)
                         + dtp * d_inner + d_conv * d_inner
                         + 3 * d_inner * d_state + d_inner * Dm)
    transcend = B * L * d_inner * (d_state + 4)
    bytes_acc = (int(hid.size) * 2 + B * Lp * Dp * 2
                 + (int(w_in_t.size) + int(w_x_t.size) + int(w_dt_t.size)
                    + int(w_out_t.size)) * 2
                 + (int(w_conv_t.size) + int(b_conv_p.size) + int(b_dt_p.size)
                    + int(neg_a.size) + int(d_skip.size)) * 4)

    def wspec(shape, mode):
        if mode is None:
            return pl.BlockSpec(shape, lambda b, t: (0, 0))
        return pl.BlockSpec(shape, lambda b, t: (0, 0), pipeline_mode=mode)

    def run(mode):
        grid_spec = pltpu.PrefetchScalarGridSpec(
            num_scalar_prefetch=0,
            grid=(B, n_lt),
            in_specs=[
                pl.BlockSpec((1, tl, Dp), lambda b, t: (b, t, 0)),
                wspec(w_in_t.shape, mode),
                wspec(w_conv_t.shape, mode),
                wspec(b_conv_p.shape, mode),
                wspec(w_x_t.shape, mode),
                wspec(w_dt_t.shape, mode),
                wspec(b_dt_p.shape, mode),
                wspec(neg_a.shape, mode),
                wspec(d_skip.shape, mode),
                wspec(w_out_t.shape, mode),
            ],
            out_specs=pl.BlockSpec((1, tl, Dp), lambda b, t: (b, t, 0)),
            scratch_shapes=[
                pltpu.VMEM((d_state, dip), _F32),
                pltpu.VMEM((d_conv - 1, dip), _F32),
                pltpu.VMEM((cpad + tl, dip), _F32),
                pltpu.VMEM((tl, dip), _F32),
                pltpu.VMEM((tl, dip), _F32),
                pltpu.VMEM((tl, d_state), _F32),
                pltpu.VMEM((tl, d_state), _F32),
                pltpu.VMEM((tl, dip), _F32),
                pltpu.VMEM((tl, dip), _F32),
                pltpu.VMEM((tl * 8, dip), _F32),
                pltpu.VMEM((tl, dip), _F32),
            ],
        )
        return pl.pallas_call(
            body,
            out_shape=jax.ShapeDtypeStruct((B, Lp, Dp), cdt),
            grid_spec=grid_spec,
            compiler_params=pltpu.CompilerParams(
                dimension_semantics=("parallel", "arbitrary"),
                vmem_limit_bytes=int(64 * 1024 * 1024 * 0.9)),
            cost_estimate=pl.CostEstimate(
                flops=int(flops), transcendentals=int(transcend),
                bytes_accessed=int(bytes_acc)),
        )(hid, w_in_t, w_conv_t, b_conv_p, w_x_t, w_dt_t, b_dt_p, neg_a,
          d_skip, w_out_t)

    try:
        out = run(pl.Buffered(1))
    except Exception:
        out = run(None)
    return out[:, :L, :Dm].astype(_F32)
